# baseline (speedup 1.0000x reference)
import functools

import jax
import jax.numpy as jnp
import numpy as np
from jax import lax
from jax.experimental import pallas as pl
from jax.experimental.pallas import tpu as pltpu
from jax.experimental.pallas import tpu_sc as plsc

ALPHA = 4
NBUF = 4


def _slow_indices(T: int) -> np.ndarray:
    n = max(1, T // ALPHA)
    return np.linspace(0.0, float(T - 1), n).astype(np.int32)


def _build_slow_gather(B, T, C, H, W, dtype, n_slow, idx):
    mesh = plsc.VectorSubcoreMesh(core_axis_name="c", subcore_axis_name="s")
    num_workers = 32
    total = B * n_slow
    per_worker = total // num_workers
    hh = H // 2
    n_chunks = per_worker * C * 2

    @functools.partial(
        pl.kernel,
        mesh=mesh,
        out_type=jax.ShapeDtypeStruct((B, n_slow, C, H, W), dtype),
        scratch_types=[
            pltpu.VMEM((NBUF, hh, W), dtype),
            pltpu.SemaphoreType.DMA,
            pltpu.SemaphoreType.DMA,
        ],
    )
    def slow_gather(in_hbm, out_hbm, buf, sem_in, sem_out):
        wid = lax.axis_index("s") * 2 + lax.axis_index("c")

        in_cp, out_cp = [], []
        for k in range(n_chunks):
            s = k // (C * 2)
            c = (k % (C * 2)) // 2
            h = k % 2
            i = wid * per_worker + s
            b = i // n_slow
            t = i % n_slow
            src_t = jnp.int32(int(idx[0]))
            for j in range(1, n_slow):
                src_t = jnp.where(t == j, jnp.int32(int(idx[j])), src_t)
            v = buf.at[k % NBUF]
            in_cp.append(pltpu.make_async_copy(
                in_hbm.at[b, src_t, c, pl.ds(h * hh, hh)], v, sem_in))
            out_cp.append(pltpu.make_async_copy(
                v, out_hbm.at[b, t, c, pl.ds(h * hh, hh)], sem_out))

        for k in range(n_chunks):
            in_cp[k].start()
            if k >= 1:
                in_cp[k - 1].wait()
                out_cp[k - 1].start()
            if k >= NBUF - 1:
                out_cp[k - (NBUF - 1)].wait()
        in_cp[n_chunks - 1].wait()
        out_cp[n_chunks - 1].start()
        for k in range(n_chunks - NBUF + 1, n_chunks):
            out_cp[k].wait()

    return slow_gather


def _build_fast_copy(shape, dtype):
    B, T, C, H, W = shape
    TB = 16
    nt = T // TB
    n = B * nt

    FBUF = 4
    chunks = [(b, t) for b in range(B) for t in range(0, T, TB)]

    def fast_body(in_hbm, out_hbm, buf, sem_in, sem_out):
        cps = []
        for k, (b, t) in enumerate(chunks):
            v = buf.at[k % FBUF]
            cps.append((
                pltpu.make_async_copy(
                    in_hbm.at[b, pl.ds(t, TB)], v, sem_in.at[k % FBUF]),
                pltpu.make_async_copy(
                    v, out_hbm.at[b, pl.ds(t, TB)], sem_out.at[k % FBUF]),
            ))
        for k in range(n):
            if k >= FBUF:
                cps[k - FBUF][1].wait()
            cps[k][0].start()
            if k >= 1:
                cps[k - 1][0].wait()
                cps[k - 1][1].start()
        cps[n - 1][0].wait()
        cps[n - 1][1].start()
        for k in range(n - FBUF, n):
            cps[k][1].wait()

    return pl.pallas_call(
        fast_body,
        out_shape=jax.ShapeDtypeStruct(shape, dtype),
        in_specs=[pl.BlockSpec(memory_space=pl.ANY)],
        out_specs=pl.BlockSpec(memory_space=pl.ANY),
        scratch_shapes=[
            pltpu.VMEM((FBUF, TB, C, H, W), dtype),
            pltpu.SemaphoreType.DMA((FBUF,)),
            pltpu.SemaphoreType.DMA((FBUF,)),
        ],
    )


def kernel(frames):
    B, T, C, H, W = frames.shape
    n_slow = max(1, T // ALPHA)
    idx = _slow_indices(T)
    slow_gather = _build_slow_gather(B, T, C, H, W, frames.dtype, n_slow, idx)
    slow_pathway = slow_gather(frames)
    fast_pathway = _build_fast_copy(frames.shape, frames.dtype)(frames)
    return (slow_pathway, fast_pathway)
